# baseline (speedup 1.0000x reference)
import functools

import jax
import jax.numpy as jnp
from jax import lax
from jax.experimental import pallas as pl
from jax.experimental.pallas import tpu as pltpu
from jax.experimental.pallas import tpu_sc as plsc

E = 16
LW = 50
LC = 100
NC = 2
NS = 16
NWORKERS = NC * NS
TCHUNK = 1024


def _tr_body(v1, vt_ref, tail_ref, out_ref,
             inb00, inb01, inb10, inb11, outb0, outb1,
             sin0, sin1, sout, sout2, str0):
    inb = ((inb00, inb01), (inb10, inb11))
    sin = (sin0, sin1)
    outb = (outb0, outb1)

    wid = lax.axis_index("s") * NC + lax.axis_index("c")

    talign = v1 // 128 * 128
    rlen = v1 - talign
    nfull = talign // TCHUNK
    tail0 = nfull * TCHUNK
    tlen = talign - tail0
    nmine = (nfull - wid + NWORKERS - 1) // NWORKERS

    lane = lax.iota(jnp.int32, E)
    lane16 = lane * E

    def start_in(c, p):
        for tr in (0, 1):
            pltpu.make_async_copy(
                vt_ref.at[pl.ds(tr * 8, 8), pl.ds(c * TCHUNK, TCHUNK)],
                inb[tr][p], sin[p]).start()

    def wait_in(p):
        for tr in (0, 1):
            pltpu.make_async_copy(
                vt_ref.at[pl.ds(0, 8), pl.ds(0, TCHUNK)],
                inb[tr][p], sin[p]).wait()

    idxs = [lane16 + (tr * 8 + r) for tr in (0, 1) for r in range(8)]

    def transpose_block(p, ngroups, vmask):
        @plsc.parallel_loop(0, ngroups, step=1, unroll=2, carry=jnp.int32(0))
        def _(tg, carry):
            ob = outb[p].at[pl.ds(tg * (E * E), E * E)]
            for tr in (0, 1):
                for r in range(8):
                    val = inb[tr][p][r, pl.ds(tg * E, E)]
                    if vmask is None:
                        plsc.store_scatter(ob, [idxs[tr * 8 + r]], val)
                    else:
                        plsc.store_scatter(ob, [idxs[tr * 8 + r]], val,
                                           mask=vmask(tg))
            return carry

    @pl.when(jnp.logical_and(wid == nfull % NWORKERS, tlen > 0))
    def _():
        for tr in (0, 1):
            pltpu.make_async_copy(
                vt_ref.at[pl.ds(tr * 8, 8), pl.ds(tail0, tlen)],
                inb[tr][0].at[:, pl.ds(0, tlen)], sin[0]).start()
        for tr in (0, 1):
            pltpu.make_async_copy(
                vt_ref.at[pl.ds(0, 8), pl.ds(0, tlen)],
                inb[tr][0].at[:, pl.ds(0, tlen)], sin[0]).wait()
        ngroups = (tlen + E - 1) // E
        transpose_block(0, ngroups,
                        lambda tg: (tg * E + lane) < tlen)
        pltpu.sync_copy(outb[0].at[pl.ds(0, tlen * E)],
                        out_ref.at[pl.ds(tail0 * E, tlen * E)])

    if rlen > 0:
        @pl.when(wid == 17)
        def _():
            pltpu.sync_copy(tail_ref, outb[1].at[pl.ds(0, rlen * E)])
            pltpu.sync_copy(outb[1].at[pl.ds(0, rlen * E)],
                            out_ref.at[pl.ds(talign * E, rlen * E)])

    souts = (sout, sout2)

    def drain_out(p):
        pltpu.make_async_copy(
            outb[p], out_ref.at[pl.ds(0, TCHUNK * E)], souts[p]).wait()

    def start_out(p, c):
        pltpu.make_async_copy(
            outb[p], out_ref.at[pl.ds(c * (TCHUNK * E), TCHUNK * E)],
            souts[p]).start()

    @pl.when(nmine > 0)
    def _():
        start_in(wid, 0)

    def outer(kk, carry):
        for p in (0, 1):
            k = kk * 2 + p
            c = wid + k * NWORKERS

            @pl.when(k < nmine)
            def _():
                wait_in(p)

                @pl.when(k + 1 < nmine)
                def _():
                    start_in(c + NWORKERS, 1 - p)

                @pl.when(k >= 2)
                def _():
                    drain_out(p)

                transpose_block(p, TCHUNK // E, None)
                start_out(p, c)
        return carry

    lax.fori_loop(0, 16, outer, 0)

    @pl.when(nmine >= 1)
    def _():
        drain_out(0)

    @pl.when(nmine >= 2)
    def _():
        drain_out(1)


def _sc_body(rpw, words_ref, ctx_ref, vocab_ref, w_ref, b_ref, out_ref,
             widx, cidx, wacc, cacc, outv, pbuf, wbv, bbv, s_idx, s_acc):
    wid = lax.axis_index("s") * NC + lax.axis_index("c")
    base0 = wid * rpw

    pltpu.sync_copy(w_ref, wbv)
    pltpu.sync_copy(b_ref, bbv)

    pltpu.make_async_copy(
        words_ref.at[:, pl.ds(base0, rpw)], widx, s_idx).start()
    pltpu.make_async_copy(
        ctx_ref.at[:, pl.ds(base0, rpw)], cidx, s_idx).start()

    z16 = jnp.zeros((E,), jnp.float32)

    def zero_body(i, carry):
        wacc[i] = z16
        cacc[i] = z16
        return carry

    lax.fori_loop(0, rpw, zero_body, 0)

    pltpu.make_async_copy(
        words_ref.at[:, pl.ds(0, rpw)], widx, s_idx).wait()

    def wg_body(j, carry):
        pltpu.async_copy(vocab_ref.at[widx.at[j]], wacc, s_acc, add=True)
        return carry

    lax.fori_loop(0, LW, wg_body, 0)

    pltpu.make_async_copy(
        ctx_ref.at[:, pl.ds(0, rpw)], cidx, s_idx).wait()

    def cg_body(j, carry):
        pltpu.async_copy(vocab_ref.at[cidx.at[j]], cacc, s_acc, add=True)
        return carry

    lax.fori_loop(0, LC, cg_body, 0)

    def drain_w(j, carry):
        pltpu.make_async_copy(vocab_ref.at[widx.at[0]], wacc, s_acc).wait()
        return carry

    lax.fori_loop(0, LW, drain_w, 0)

    def drain_c(j, carry):
        pltpu.make_async_copy(vocab_ref.at[cidx.at[0]], cacc, s_acc).wait()
        return carry

    lax.fori_loop(0, LC, drain_c, 0)

    wv = wbv[...]
    bv = bbv[...]
    lane = lax.iota(jnp.int32, E)
    gbase = lane * E

    def group_body(g, carry):
        def prod_body(i, carry2):
            r = g * E + i
            pbuf[pl.ds(i * E, E)] = wacc[r] * cacc[r]
            return carry2

        lax.fori_loop(0, E, prod_body, 0)

        acc = z16
        for c in range(E):
            acc = acc + plsc.load_gather(pbuf, [gbase + c])
        zv = acc * wv + bv
        ov = 1.0 / (1.0 + jnp.exp(-zv))
        outv[pl.ds(g * E, E)] = ov
        return carry

    lax.fori_loop(0, rpw // E, group_body, 0)

    pltpu.sync_copy(outv, out_ref.at[pl.ds(base0, rpw)])


@functools.partial(jax.jit, static_argnames=())
def kernel(words, context, vocab, dense_w, dense_b):
    B, lw = words.shape
    lc = context.shape[1] * context.shape[2]
    assert lw == LW and lc == LC and vocab.shape[1] == E
    assert B % (NWORKERS * E) == 0
    rpw = B // NWORKERS

    words_t = jnp.asarray(words, jnp.int32).T
    ctx_t = jnp.asarray(context, jnp.int32).transpose(2, 1, 0).reshape(LC, B)
    vocab = jnp.asarray(vocab, jnp.float32)
    w16 = jnp.broadcast_to(
        jnp.asarray(dense_w, jnp.float32).reshape(-1)[:1], (E,))
    b16 = jnp.broadcast_to(
        jnp.asarray(dense_b, jnp.float32).reshape(-1)[:1], (E,))

    mesh = plsc.VectorSubcoreMesh(
        core_axis_name="c", subcore_axis_name="s",
        num_cores=NC, num_subcores=NS)

    V = vocab.shape[0]
    vt = vocab.T
    run_tr = pl.kernel(
        functools.partial(_tr_body, V),
        out_type=jax.ShapeDtypeStruct((V * E,), jnp.float32),
        mesh=mesh,
        compiler_params=pltpu.CompilerParams(needs_layout_passes=False),
        scratch_types=[
            pltpu.VMEM((8, TCHUNK), jnp.float32),
            pltpu.VMEM((8, TCHUNK), jnp.float32),
            pltpu.VMEM((8, TCHUNK), jnp.float32),
            pltpu.VMEM((8, TCHUNK), jnp.float32),
            pltpu.VMEM((TCHUNK * E,), jnp.float32),
            pltpu.VMEM((TCHUNK * E,), jnp.float32),
            pltpu.SemaphoreType.DMA,
            pltpu.SemaphoreType.DMA,
            pltpu.SemaphoreType.DMA,
            pltpu.SemaphoreType.DMA,
            pltpu.SemaphoreType.DMA,
        ],
    )
    rump0 = V // 128 * 128
    tail_flat = lax.slice(vocab, (rump0, 0), (V, E)).reshape(-1)
    table = run_tr(vt, tail_flat).reshape(V, E)

    run = pl.kernel(
        functools.partial(_sc_body, rpw),
        out_type=jax.ShapeDtypeStruct((B,), jnp.float32),
        mesh=mesh,
        compiler_params=pltpu.CompilerParams(
            needs_layout_passes=False, use_tc_tiling_on_sc=False),
        scratch_types=[
            pltpu.VMEM((LW, rpw), jnp.int32),
            pltpu.VMEM((LC, rpw), jnp.int32),
            pltpu.VMEM((rpw, E), jnp.float32),
            pltpu.VMEM((rpw, E), jnp.float32),
            pltpu.VMEM((rpw,), jnp.float32),
            pltpu.VMEM((E * E,), jnp.float32),
            pltpu.VMEM((E,), jnp.float32),
            pltpu.VMEM((E,), jnp.float32),
            pltpu.SemaphoreType.DMA,
            pltpu.SemaphoreType.DMA,
        ],
    )
    out = run(words_t, ctx_t, table, w16, b16)
    return out.reshape(B, 1)


# speedup vs baseline: 1.3552x; 1.3552x over previous
import functools

import jax
import jax.numpy as jnp
from jax import lax
from jax.experimental import pallas as pl
from jax.experimental.pallas import tpu as pltpu
from jax.experimental.pallas import tpu_sc as plsc

E = 16
LW = 50
LC = 100
NC = 2
NS = 16
NWORKERS = NC * NS
TCHUNK = 2048


def _tr_body(v1, vt_ref, tail_ref, out_ref,
             inb00, inb01, inb10, inb11, outb0, outb1,
             sin0, sin1, sout, sout2, str0):
    inb = ((inb00, inb01), (inb10, inb11))
    sin = (sin0, sin1)
    outb = (outb0, outb1)

    wid = lax.axis_index("s") * NC + lax.axis_index("c")

    talign = v1 // 128 * 128
    rlen = v1 - talign
    nfull = talign // TCHUNK
    tail0 = nfull * TCHUNK
    tlen = talign - tail0
    nmine = (nfull - wid + NWORKERS - 1) // NWORKERS

    lane = lax.iota(jnp.int32, E)
    lane16 = lane * E

    def start_in(c, p):
        for tr in (0, 1):
            pltpu.make_async_copy(
                vt_ref.at[pl.ds(tr * 8, 8), pl.ds(c * TCHUNK, TCHUNK)],
                inb[tr][p], sin[p]).start()

    def wait_in(p):
        for tr in (0, 1):
            pltpu.make_async_copy(
                vt_ref.at[pl.ds(0, 8), pl.ds(0, TCHUNK)],
                inb[tr][p], sin[p]).wait()

    idxs = [lane16 + (tr * 8 + r) for tr in (0, 1) for r in range(8)]

    def transpose_block(p, ngroups, vmask):
        @plsc.parallel_loop(0, ngroups, step=1, carry=jnp.int32(0))
        def _(tg, carry):
            ob = outb[p].at[pl.ds(tg * (E * E), E * E)]
            for tr in (0, 1):
                for r in range(8):
                    val = inb[tr][p][r, pl.ds(tg * E, E)]
                    if vmask is None:
                        plsc.store_scatter(ob, [idxs[tr * 8 + r]], val)
                    else:
                        plsc.store_scatter(ob, [idxs[tr * 8 + r]], val,
                                           mask=vmask(tg))
            return carry

    @pl.when(jnp.logical_and(wid == nfull % NWORKERS, tlen > 0))
    def _():
        for tr in (0, 1):
            pltpu.make_async_copy(
                vt_ref.at[pl.ds(tr * 8, 8), pl.ds(tail0, tlen)],
                inb[tr][0].at[:, pl.ds(0, tlen)], sin[0]).start()
        for tr in (0, 1):
            pltpu.make_async_copy(
                vt_ref.at[pl.ds(0, 8), pl.ds(0, tlen)],
                inb[tr][0].at[:, pl.ds(0, tlen)], sin[0]).wait()
        ngroups = (tlen + E - 1) // E
        transpose_block(0, ngroups,
                        lambda tg: (tg * E + lane) < tlen)
        pltpu.sync_copy(outb[0].at[pl.ds(0, tlen * E)],
                        out_ref.at[pl.ds(tail0 * E, tlen * E)])

    if rlen > 0:
        @pl.when(wid == 17)
        def _():
            pltpu.sync_copy(tail_ref, outb[1].at[pl.ds(0, rlen * E)])
            pltpu.sync_copy(outb[1].at[pl.ds(0, rlen * E)],
                            out_ref.at[pl.ds(talign * E, rlen * E)])

    souts = (sout, sout2)

    def drain_out(p):
        pltpu.make_async_copy(
            outb[p], out_ref.at[pl.ds(0, TCHUNK * E)], souts[p]).wait()

    def start_out(p, c):
        pltpu.make_async_copy(
            outb[p], out_ref.at[pl.ds(c * (TCHUNK * E), TCHUNK * E)],
            souts[p]).start()

    @pl.when(nmine > 0)
    def _():
        start_in(wid, 0)

    def outer(kk, carry):
        for p in (0, 1):
            k = kk * 2 + p
            c = wid + k * NWORKERS

            @pl.when(k < nmine)
            def _():
                wait_in(p)

                @pl.when(k + 1 < nmine)
                def _():
                    start_in(c + NWORKERS, 1 - p)

                @pl.when(k >= 2)
                def _():
                    drain_out(p)

                transpose_block(p, TCHUNK // E, None)
                start_out(p, c)
        return carry

    lax.fori_loop(0, 16, outer, 0)

    @pl.when(nmine >= 1)
    def _():
        drain_out(0)

    @pl.when(nmine >= 2)
    def _():
        drain_out(1)


def _sc_body(rpw, words_ref, ctx_ref, vocab_ref, w_ref, b_ref, out_ref,
             widx, cidx, wacc, cacc, outv, pbuf, wbv, bbv, s_idx, s_acc):
    wid = lax.axis_index("s") * NC + lax.axis_index("c")
    base0 = wid * rpw

    pltpu.sync_copy(w_ref, wbv)
    pltpu.sync_copy(b_ref, bbv)

    pltpu.make_async_copy(
        words_ref.at[:, pl.ds(base0, rpw)], widx, s_idx).start()
    pltpu.make_async_copy(
        ctx_ref.at[:, pl.ds(base0, rpw)], cidx, s_idx).start()

    z16 = jnp.zeros((E,), jnp.float32)

    def zero_body(i, carry):
        wacc[i] = z16
        cacc[i] = z16
        return carry

    lax.fori_loop(0, rpw, zero_body, 0)

    pltpu.make_async_copy(
        words_ref.at[:, pl.ds(0, rpw)], widx, s_idx).wait()

    def wg_body(j, carry):
        pltpu.async_copy(vocab_ref.at[widx.at[j]], wacc, s_acc, add=True)
        return carry

    lax.fori_loop(0, LW, wg_body, 0)

    pltpu.make_async_copy(
        ctx_ref.at[:, pl.ds(0, rpw)], cidx, s_idx).wait()

    def cg_body(j, carry):
        pltpu.async_copy(vocab_ref.at[cidx.at[j]], cacc, s_acc, add=True)
        return carry

    lax.fori_loop(0, LC, cg_body, 0)

    def drain_w(j, carry):
        pltpu.make_async_copy(vocab_ref.at[widx.at[0]], wacc, s_acc).wait()
        return carry

    lax.fori_loop(0, LW, drain_w, 0)

    def drain_c(j, carry):
        pltpu.make_async_copy(vocab_ref.at[cidx.at[0]], cacc, s_acc).wait()
        return carry

    lax.fori_loop(0, LC, drain_c, 0)

    wv = wbv[...]
    bv = bbv[...]
    lane = lax.iota(jnp.int32, E)
    gbase = lane * E

    def group_body(g, carry):
        def prod_body(i, carry2):
            r = g * E + i
            pbuf[pl.ds(i * E, E)] = wacc[r] * cacc[r]
            return carry2

        lax.fori_loop(0, E, prod_body, 0)

        acc = z16
        for c in range(E):
            acc = acc + plsc.load_gather(pbuf, [gbase + c])
        zv = acc * wv + bv
        ov = 1.0 / (1.0 + jnp.exp(-zv))
        outv[pl.ds(g * E, E)] = ov
        return carry

    lax.fori_loop(0, rpw // E, group_body, 0)

    pltpu.sync_copy(outv, out_ref.at[pl.ds(base0, rpw)])


@functools.partial(jax.jit, static_argnames=())
def kernel(words, context, vocab, dense_w, dense_b):
    B, lw = words.shape
    lc = context.shape[1] * context.shape[2]
    assert lw == LW and lc == LC and vocab.shape[1] == E
    assert B % (NWORKERS * E) == 0
    rpw = B // NWORKERS

    words_t = jnp.asarray(words, jnp.int32).T
    ctx_t = jnp.asarray(context, jnp.int32).transpose(2, 1, 0).reshape(LC, B)
    vocab = jnp.asarray(vocab, jnp.float32)
    w16 = jnp.broadcast_to(
        jnp.asarray(dense_w, jnp.float32).reshape(-1)[:1], (E,))
    b16 = jnp.broadcast_to(
        jnp.asarray(dense_b, jnp.float32).reshape(-1)[:1], (E,))

    mesh = plsc.VectorSubcoreMesh(
        core_axis_name="c", subcore_axis_name="s",
        num_cores=NC, num_subcores=NS)

    V = vocab.shape[0]
    vt = vocab.T
    run_tr = pl.kernel(
        functools.partial(_tr_body, V),
        out_type=jax.ShapeDtypeStruct((V * E,), jnp.float32),
        mesh=mesh,
        compiler_params=pltpu.CompilerParams(needs_layout_passes=False),
        scratch_types=[
            pltpu.VMEM((8, TCHUNK), jnp.float32),
            pltpu.VMEM((8, TCHUNK), jnp.float32),
            pltpu.VMEM((8, TCHUNK), jnp.float32),
            pltpu.VMEM((8, TCHUNK), jnp.float32),
            pltpu.VMEM((TCHUNK * E,), jnp.float32),
            pltpu.VMEM((TCHUNK * E,), jnp.float32),
            pltpu.SemaphoreType.DMA,
            pltpu.SemaphoreType.DMA,
            pltpu.SemaphoreType.DMA,
            pltpu.SemaphoreType.DMA,
            pltpu.SemaphoreType.DMA,
        ],
    )
    rump0 = V // 128 * 128
    tail_flat = lax.slice(vocab, (rump0, 0), (V, E)).reshape(-1)
    table = run_tr(vt, tail_flat).reshape(V, E)

    run = pl.kernel(
        functools.partial(_sc_body, rpw),
        out_type=jax.ShapeDtypeStruct((B,), jnp.float32),
        mesh=mesh,
        compiler_params=pltpu.CompilerParams(
            needs_layout_passes=False, use_tc_tiling_on_sc=False),
        scratch_types=[
            pltpu.VMEM((LW, rpw), jnp.int32),
            pltpu.VMEM((LC, rpw), jnp.int32),
            pltpu.VMEM((rpw, E), jnp.float32),
            pltpu.VMEM((rpw, E), jnp.float32),
            pltpu.VMEM((rpw,), jnp.float32),
            pltpu.VMEM((E * E,), jnp.float32),
            pltpu.VMEM((E,), jnp.float32),
            pltpu.VMEM((E,), jnp.float32),
            pltpu.SemaphoreType.DMA,
            pltpu.SemaphoreType.DMA,
        ],
    )
    out = run(words_t, ctx_t, table, w16, b16)
    return out.reshape(B, 1)


# speedup vs baseline: 1.3564x; 1.0008x over previous
import functools

import jax
import jax.numpy as jnp
from jax import lax
from jax.experimental import pallas as pl
from jax.experimental.pallas import tpu as pltpu
from jax.experimental.pallas import tpu_sc as plsc

E = 16
LW = 50
LC = 100
NC = 2
NS = 16
NWORKERS = NC * NS
TCHUNK = 2048


def _tr_body(v1, vt_ref, tail_ref, out_ref,
             inb00, inb01, inb10, inb11, outb0, outb1,
             sin0, sin1, sout, sout2):
    inb = ((inb00, inb01), (inb10, inb11))
    sin = (sin0, sin1)
    outb = (outb0, outb1)

    wid = lax.axis_index("s") * NC + lax.axis_index("c")

    talign = v1 // 128 * 128
    rlen = v1 - talign
    nfull = talign // TCHUNK
    tail0 = nfull * TCHUNK
    tlen = talign - tail0
    nmine = (nfull - wid + NWORKERS - 1) // NWORKERS

    lane = lax.iota(jnp.int32, E)
    lane16 = lane * E

    def start_in(c, p):
        for tr in (0, 1):
            pltpu.make_async_copy(
                vt_ref.at[pl.ds(tr * 8, 8), pl.ds(c * TCHUNK, TCHUNK)],
                inb[tr][p], sin[p]).start()

    def wait_in(p):
        for tr in (0, 1):
            pltpu.make_async_copy(
                vt_ref.at[pl.ds(0, 8), pl.ds(0, TCHUNK)],
                inb[tr][p], sin[p]).wait()

    idxs = [lane16 + (tr * 8 + r) for tr in (0, 1) for r in range(8)]

    def transpose_block(p, ngroups, vmask):
        @plsc.parallel_loop(0, ngroups, step=1, carry=jnp.int32(0))
        def _(tg, carry):
            ob = outb[p].at[pl.ds(tg * (E * E), E * E)]
            for tr in (0, 1):
                for r in range(8):
                    val = inb[tr][p][r, pl.ds(tg * E, E)]
                    if vmask is None:
                        plsc.store_scatter(ob, [idxs[tr * 8 + r]], val)
                    else:
                        plsc.store_scatter(ob, [idxs[tr * 8 + r]], val,
                                           mask=vmask(tg))
            return carry

    @pl.when(jnp.logical_and(wid == nfull % NWORKERS, tlen > 0))
    def _():
        for tr in (0, 1):
            pltpu.make_async_copy(
                vt_ref.at[pl.ds(tr * 8, 8), pl.ds(tail0, tlen)],
                inb[tr][0].at[:, pl.ds(0, tlen)], sin[0]).start()
        for tr in (0, 1):
            pltpu.make_async_copy(
                vt_ref.at[pl.ds(0, 8), pl.ds(0, tlen)],
                inb[tr][0].at[:, pl.ds(0, tlen)], sin[0]).wait()
        ngroups = (tlen + E - 1) // E
        transpose_block(0, ngroups,
                        lambda tg: (tg * E + lane) < tlen)
        pltpu.sync_copy(outb[0].at[pl.ds(0, tlen * E)],
                        out_ref.at[pl.ds(tail0 * E, tlen * E)])

    if rlen > 0:
        @pl.when(wid == 17)
        def _():
            pltpu.sync_copy(tail_ref, outb[1].at[pl.ds(0, rlen * E)])
            pltpu.sync_copy(outb[1].at[pl.ds(0, rlen * E)],
                            out_ref.at[pl.ds(talign * E, rlen * E)])

    souts = (sout, sout2)

    def drain_out(p):
        pltpu.make_async_copy(
            outb[p], out_ref.at[pl.ds(0, TCHUNK * E)], souts[p]).wait()

    def start_out(p, c):
        pltpu.make_async_copy(
            outb[p], out_ref.at[pl.ds(c * (TCHUNK * E), TCHUNK * E)],
            souts[p]).start()

    @pl.when(nmine > 0)
    def _():
        start_in(wid, 0)

    def outer(kk, carry):
        for p in (0, 1):
            k = kk * 2 + p
            c = wid + k * NWORKERS

            @pl.when(k < nmine)
            def _():
                wait_in(p)

                @pl.when(k + 1 < nmine)
                def _():
                    start_in(c + NWORKERS, 1 - p)

                @pl.when(k >= 2)
                def _():
                    drain_out(p)

                transpose_block(p, TCHUNK // E, None)
                start_out(p, c)
        return carry

    lax.fori_loop(0, 16, outer, 0)

    @pl.when(nmine >= 1)
    def _():
        drain_out(0)

    @pl.when(nmine >= 2)
    def _():
        drain_out(1)


def _sc_body(rpw, words_ref, ctx_ref, vocab_ref, w_ref, b_ref, out_ref,
             widx, cidx, wacc, cacc, outv, pbuf, wbv, bbv, s_idx, s_acc):
    wid = lax.axis_index("s") * NC + lax.axis_index("c")
    base0 = wid * rpw

    pltpu.sync_copy(w_ref, wbv)
    pltpu.sync_copy(b_ref, bbv)

    pltpu.make_async_copy(
        words_ref.at[:, pl.ds(base0, rpw)], widx, s_idx).start()
    pltpu.make_async_copy(
        ctx_ref.at[:, pl.ds(base0, rpw)], cidx, s_idx).start()

    z16 = jnp.zeros((E,), jnp.float32)

    def zero_body(i, carry):
        wacc[i] = z16
        cacc[i] = z16
        return carry

    lax.fori_loop(0, rpw, zero_body, 0)

    pltpu.make_async_copy(
        words_ref.at[:, pl.ds(0, rpw)], widx, s_idx).wait()

    def wg_body(j, carry):
        pltpu.async_copy(vocab_ref.at[widx.at[j]], wacc, s_acc, add=True)
        return carry

    lax.fori_loop(0, LW, wg_body, 0)

    pltpu.make_async_copy(
        ctx_ref.at[:, pl.ds(0, rpw)], cidx, s_idx).wait()

    def cg_body(j, carry):
        pltpu.async_copy(vocab_ref.at[cidx.at[j]], cacc, s_acc, add=True)
        return carry

    lax.fori_loop(0, LC, cg_body, 0)

    def drain_w(j, carry):
        pltpu.make_async_copy(vocab_ref.at[widx.at[0]], wacc, s_acc).wait()
        return carry

    lax.fori_loop(0, LW, drain_w, 0)

    def drain_c(j, carry):
        pltpu.make_async_copy(vocab_ref.at[cidx.at[0]], cacc, s_acc).wait()
        return carry

    lax.fori_loop(0, LC, drain_c, 0)

    wv = wbv[...]
    bv = bbv[...]
    lane = lax.iota(jnp.int32, E)
    gbase = lane * E

    def group_body(g, carry):
        def prod_body(i, carry2):
            r = g * E + i
            pbuf[pl.ds(i * E, E)] = wacc[r] * cacc[r]
            return carry2

        lax.fori_loop(0, E, prod_body, 0)

        acc = z16
        for c in range(E):
            acc = acc + plsc.load_gather(pbuf, [gbase + c])
        zv = acc * wv + bv
        ov = 1.0 / (1.0 + jnp.exp(-zv))
        outv[pl.ds(g * E, E)] = ov
        return carry

    lax.fori_loop(0, rpw // E, group_body, 0)

    pltpu.sync_copy(outv, out_ref.at[pl.ds(base0, rpw)])


@functools.partial(jax.jit, static_argnames=())
def kernel(words, context, vocab, dense_w, dense_b):
    B, lw = words.shape
    lc = context.shape[1] * context.shape[2]
    assert lw == LW and lc == LC and vocab.shape[1] == E
    assert B % (NWORKERS * E) == 0
    rpw = B // NWORKERS

    words_t = jnp.asarray(words, jnp.int32).T
    ctx_t = jnp.asarray(context, jnp.int32).transpose(2, 1, 0).reshape(LC, B)
    vocab = jnp.asarray(vocab, jnp.float32)
    w16 = jnp.broadcast_to(
        jnp.asarray(dense_w, jnp.float32).reshape(-1)[:1], (E,))
    b16 = jnp.broadcast_to(
        jnp.asarray(dense_b, jnp.float32).reshape(-1)[:1], (E,))

    mesh = plsc.VectorSubcoreMesh(
        core_axis_name="c", subcore_axis_name="s",
        num_cores=NC, num_subcores=NS)

    V = vocab.shape[0]
    vt = vocab.T
    run_tr = pl.kernel(
        functools.partial(_tr_body, V),
        out_type=jax.ShapeDtypeStruct((V * E,), jnp.float32),
        mesh=mesh,
        compiler_params=pltpu.CompilerParams(needs_layout_passes=False),
        scratch_types=[
            pltpu.VMEM((8, TCHUNK), jnp.float32),
            pltpu.VMEM((8, TCHUNK), jnp.float32),
            pltpu.VMEM((8, TCHUNK), jnp.float32),
            pltpu.VMEM((8, TCHUNK), jnp.float32),
            pltpu.VMEM((TCHUNK * E,), jnp.float32),
            pltpu.VMEM((TCHUNK * E,), jnp.float32),
            pltpu.SemaphoreType.DMA,
            pltpu.SemaphoreType.DMA,
            pltpu.SemaphoreType.DMA,
            pltpu.SemaphoreType.DMA,
        ],
    )
    rump0 = V // 128 * 128
    tail_flat = lax.slice(vocab, (rump0, 0), (V, E)).reshape(-1)
    table = run_tr(vt, tail_flat).reshape(V, E)

    run = pl.kernel(
        functools.partial(_sc_body, rpw),
        out_type=jax.ShapeDtypeStruct((B,), jnp.float32),
        mesh=mesh,
        compiler_params=pltpu.CompilerParams(
            needs_layout_passes=False, use_tc_tiling_on_sc=False),
        scratch_types=[
            pltpu.VMEM((LW, rpw), jnp.int32),
            pltpu.VMEM((LC, rpw), jnp.int32),
            pltpu.VMEM((rpw, E), jnp.float32),
            pltpu.VMEM((rpw, E), jnp.float32),
            pltpu.VMEM((rpw,), jnp.float32),
            pltpu.VMEM((E * E,), jnp.float32),
            pltpu.VMEM((E,), jnp.float32),
            pltpu.VMEM((E,), jnp.float32),
            pltpu.SemaphoreType.DMA,
            pltpu.SemaphoreType.DMA,
        ],
    )
    out = run(words_t, ctx_t, table, w16, b16)
    return out.reshape(B, 1)
